# baseline (speedup 1.0000x reference)
import functools

import jax
import jax.numpy as jnp
from jax import lax
from jax.experimental import pallas as pl
from jax.experimental.pallas import tpu as pltpu
from jax.experimental.pallas import tpu_sc as plsc

N = 10000
D = 128
E = 320000
NC = 2
NS = 16
NW = NC * NS
EPW = E // NW
CH = 128
NCHT = E // CH
NCHW = NCHT // NW
NEXTRA = NCHT - NCHW * NW
NPAD = 10240
RPW = NPAD // NS
ZROWS = 40
CPROWS = 160


def _tc_pre_body(x_ref, c_ref, o_ref):
    x = x_ref[...]
    cs = jnp.sqrt(jnp.abs(c_ref[...]))
    xn = jnp.sqrt(jnp.sum(x * x, axis=1, keepdims=True))
    xn = jnp.maximum(xn, 1e-15)
    y = jnp.clip(cs * xn, -1.0 + 1e-7, 1.0 - 1e-7)
    at = 0.5 * jnp.log((1.0 + y) / (1.0 - y))
    o_ref[...] = at * x / (cs * xn)


def _tc_pre(x, c_param):
    br = 400
    return pl.pallas_call(
        _tc_pre_body,
        grid=(N // br,),
        in_specs=[
            pl.BlockSpec((br, D), lambda i: (i, 0)),
            pl.BlockSpec((1,), lambda i: (0,)),
        ],
        out_specs=pl.BlockSpec((br, D), lambda i: (i, 0)),
        out_shape=jax.ShapeDtypeStruct((N, D), jnp.float32),
    )(x, c_param)


def _sc_agg_body(tang_hbm, edge_hbm, out_hbm, deg_hbm,
                 idxb, rowsb, zero_v, hist_v, acc_sh,
                 isems, gsems, ssems, zsem):
    cid = lax.axis_index("c")
    sid = lax.axis_index("s")
    wid = cid * NS + sid
    row0 = sid * RPW
    c0 = wid * NCHW

    def idxcopy(k, ib):
        off = pl.multiple_of((c0 + k) * CH, CH)
        return pltpu.make_async_copy(edge_hbm.at[:, pl.ds(off, CH)],
                                     idxb[ib], isems[ib])

    def gather(ib, b):
        return pltpu.make_async_copy(tang_hbm.at[idxb[ib].at[0]],
                                     rowsb[b], gsems[b])

    def scatter(ib, b):
        return pltpu.make_async_copy(rowsb[b], acc_sh.at[idxb[ib].at[1]],
                                     ssems[b])

    idxcopy(0, 0).start()
    idxcopy(1, 1).start()
    idxcopy(0, 0).wait()
    gather(0, 0).start()

    def zbody(i, _):
        r = i // (D // 16)
        c0 = (i % (D // 16)) * 16
        zero_v[r, pl.ds(c0, 16)] = jnp.zeros((16,), jnp.float32)
        return 0
    lax.fori_loop(0, ZROWS * (D // 16), zbody, 0)

    def zfire(k, _):
        pltpu.async_copy(zero_v, acc_sh.at[pl.ds(row0 + k * ZROWS, ZROWS)],
                         zsem)
        return 0
    lax.fori_loop(0, RPW // ZROWS, zfire, 0)

    def hzero(i, _):
        hist_v[pl.ds(i * 16, 16)] = jnp.zeros((16,), jnp.float32)
        return 0
    lax.fori_loop(0, NPAD // 16, hzero, 0)

    def zdrain(k, _):
        pltpu.make_async_copy(
            zero_v, acc_sh.at[pl.ds(row0 + k * ZROWS, ZROWS)], zsem).wait()
        return 0
    lax.fori_loop(0, RPW // ZROWS, zdrain, 0)

    plsc.subcore_barrier()

    ones16 = jnp.ones((16,), jnp.float32)

    def hist(ib):
        for j in range(CH // 16):
            dv = idxb[ib][1, pl.ds(j * 16, 16)]
            plsc.addupdate_scatter(hist_v, [dv], ones16)

    def step(k, j):
        ib, b = j, j % 2
        gather(ib, b).wait()
        scatter(ib, b).start(add=True)
        hist(ib)

        @pl.when(k >= 1)
        def _():
            scatter((j + 3) % 4, (j + 1) % 2).wait()

        @pl.when(k + 1 < NCHW)
        def _():
            idxcopy(k + 1, (j + 1) % 4).wait()
            gather((j + 1) % 4, (j + 1) % 2).start()

        @pl.when(k + 2 < NCHW)
        def _():
            idxcopy(k + 2, (j + 2) % 4).start()

    def mbody(kk, _):
        for j in range(4):
            step(4 * kk + j, j)
        return 0
    lax.fori_loop(0, NCHW // 4, mbody, 0)
    step(NCHW - 2, (NCHW - 2) % 4)
    step(NCHW - 1, (NCHW - 1) % 4)

    scatter((NCHW - 1) % 4, (NCHW - 1) % 2).wait()

    @pl.when(wid < NEXTRA)
    def _():
        ke = NW * NCHW + wid - c0
        idxcopy(ke, 0).start()
        idxcopy(ke, 0).wait()
        gather(0, 0).start()
        gather(0, 0).wait()
        scatter(0, 0).start(add=True)
        hist(0)
        scatter(0, 0).wait()

    pltpu.sync_copy(hist_v, deg_hbm.at[wid])

    plsc.subcore_barrier()

    def ofire(k, _):
        r = row0 + k * CPROWS
        pltpu.async_copy(acc_sh.at[pl.ds(r, CPROWS)],
                         out_hbm.at[cid, pl.ds(r, CPROWS)], zsem)
        return 0
    lax.fori_loop(0, RPW // CPROWS, ofire, 0)

    def odrain(k, _):
        r = row0 + k * CPROWS
        pltpu.make_async_copy(acc_sh.at[pl.ds(r, CPROWS)],
                              out_hbm.at[cid, pl.ds(r, CPROWS)], zsem).wait()
        return 0
    lax.fori_loop(0, RPW // CPROWS, odrain, 0)


def _sc_agg(tangent, edge_index):
    mesh = plsc.VectorSubcoreMesh(core_axis_name="c", subcore_axis_name="s")
    f = functools.partial(
        pl.kernel,
        out_type=(
            jax.ShapeDtypeStruct((NC, NPAD, D), jnp.float32),
            jax.ShapeDtypeStruct((NW, NPAD), jnp.float32),
        ),
        mesh=mesh,
        scratch_types=[
            [pltpu.VMEM((2, CH), jnp.int32) for _ in range(4)],
            [pltpu.VMEM((CH, D), jnp.float32) for _ in range(2)],
            pltpu.VMEM((ZROWS, D), jnp.float32),
            pltpu.VMEM((NPAD,), jnp.float32),
            pltpu.VMEM_SHARED((NPAD, D), jnp.float32),
            [pltpu.SemaphoreType.DMA for _ in range(4)],
            [pltpu.SemaphoreType.DMA for _ in range(2)],
            [pltpu.SemaphoreType.DMA for _ in range(2)],
            pltpu.SemaphoreType.DMA,
        ],
        compiler_params=pltpu.CompilerParams(needs_layout_passes=False),
    )(_sc_agg_body)
    return f(tangent, edge_index)


def _tc_post_body(p_ref, deg_ref, w_ref, b_ref, c_ref, o_ref):
    agg_t = p_ref[0] + p_ref[1]
    ones = jnp.ones((NW, 1), jnp.float32)
    deg = lax.dot_general(
        deg_ref[...], ones,
        dimension_numbers=(((0,), (0,)), ((), ())),
        preferred_element_type=jnp.float32,
        precision=lax.Precision.HIGHEST,
    )
    agg = lax.dot_general(
        agg_t, w_ref[...],
        dimension_numbers=(((1,), (1,)), ((), ())),
        preferred_element_type=jnp.float32,
        precision=lax.Precision.HIGHEST,
    ) + deg * b_ref[...].reshape(1, D)
    neigh = agg / jnp.maximum(deg, 1.0)
    vn = jnp.sqrt(jnp.sum(neigh * neigh, axis=1, keepdims=True))
    vn = jnp.maximum(vn, 1e-15)
    cs = jnp.sqrt(jnp.abs(c_ref[...]))
    arg = cs * vn
    o_ref[...] = jnp.tanh(arg) * neigh / arg


def _tc_post(partials, degs, W, b, c_param):
    br = 256
    return pl.pallas_call(
        _tc_post_body,
        grid=(NPAD // br,),
        in_specs=[
            pl.BlockSpec((NC, br, D), lambda i: (0, i, 0)),
            pl.BlockSpec((NW, br), lambda i: (0, i)),
            pl.BlockSpec((D, D), lambda i: (0, 0)),
            pl.BlockSpec((D,), lambda i: (0,)),
            pl.BlockSpec((1,), lambda i: (0,)),
        ],
        out_specs=pl.BlockSpec((br, D), lambda i: (i, 0)),
        out_shape=jax.ShapeDtypeStruct((N, D), jnp.float32),
    )(partials, degs, W, b, c_param)


def kernel(x, edge_index, W, b, c_param):
    tangent = _tc_pre(x, c_param)
    partials, degs = _sc_agg(tangent, edge_index)
    return _tc_post(partials, degs, W, b, c_param)


# speedup vs baseline: 1.0898x; 1.0898x over previous
import functools

import jax
import jax.numpy as jnp
from jax import lax
from jax.experimental import pallas as pl
from jax.experimental.pallas import tpu as pltpu
from jax.experimental.pallas import tpu_sc as plsc

N = 10000
D = 128
E = 320000
NC = 2
NS = 16
NW = NC * NS
EPW = E // NW
CH = 128
NCHT = E // CH
NCHW = NCHT // NW
NEXTRA = NCHT - NCHW * NW
NPAD = 10240
RPW = NPAD // NS
ZROWS = 40
CPROWS = 160


def _tc_pre_body(x_ref, c_ref, o_ref):
    x = x_ref[...]
    cs = jnp.sqrt(jnp.abs(c_ref[...]))
    xn = jnp.sqrt(jnp.sum(x * x, axis=1, keepdims=True))
    xn = jnp.maximum(xn, 1e-15)
    y = jnp.clip(cs * xn, -1.0 + 1e-7, 1.0 - 1e-7)
    at = 0.5 * jnp.log((1.0 + y) / (1.0 - y))
    o_ref[...] = at * x / (cs * xn)


def _tc_pre(x, c_param):
    br = 1000
    return pl.pallas_call(
        _tc_pre_body,
        grid=(N // br,),
        in_specs=[
            pl.BlockSpec((br, D), lambda i: (i, 0)),
            pl.BlockSpec((1,), lambda i: (0,)),
        ],
        out_specs=pl.BlockSpec((br, D), lambda i: (i, 0)),
        out_shape=jax.ShapeDtypeStruct((N, D), jnp.float32),
    )(x, c_param)


def _sc_agg_body(tang_hbm, edge_hbm, out_hbm, deg_hbm,
                 idxb, rowsb, zero_v, hist_v, acc_sh,
                 isems, gsems, ssems, zsem):
    cid = lax.axis_index("c")
    sid = lax.axis_index("s")
    wid = cid * NS + sid
    row0 = sid * RPW
    c0 = wid * NCHW

    def idxcopy(k, ib):
        off = pl.multiple_of((c0 + k) * CH, CH)
        return pltpu.make_async_copy(edge_hbm.at[:, pl.ds(off, CH)],
                                     idxb[ib], isems[ib])

    def gather(ib, b):
        return pltpu.make_async_copy(tang_hbm.at[idxb[ib].at[0]],
                                     rowsb[b], gsems[b])

    def scatter(ib, b):
        return pltpu.make_async_copy(rowsb[b], acc_sh.at[idxb[ib].at[1]],
                                     ssems[b])

    idxcopy(0, 0).start()
    idxcopy(1, 1).start()
    idxcopy(0, 0).wait()
    gather(0, 0).start()

    def zbody(i, _):
        r = i // (D // 16)
        c0 = (i % (D // 16)) * 16
        zero_v[r, pl.ds(c0, 16)] = jnp.zeros((16,), jnp.float32)
        return 0
    lax.fori_loop(0, ZROWS * (D // 16), zbody, 0)

    def zfire(k, _):
        pltpu.async_copy(zero_v, acc_sh.at[pl.ds(row0 + k * ZROWS, ZROWS)],
                         zsem)
        return 0
    lax.fori_loop(0, RPW // ZROWS, zfire, 0)

    def hzero(i, _):
        hist_v[pl.ds(i * 16, 16)] = jnp.zeros((16,), jnp.float32)
        return 0
    lax.fori_loop(0, NPAD // 16, hzero, 0)

    def zdrain(k, _):
        pltpu.make_async_copy(
            zero_v, acc_sh.at[pl.ds(row0 + k * ZROWS, ZROWS)], zsem).wait()
        return 0
    lax.fori_loop(0, RPW // ZROWS, zdrain, 0)

    plsc.subcore_barrier()

    ones16 = jnp.ones((16,), jnp.float32)

    def hist(ib):
        for j in range(CH // 16):
            dv = idxb[ib][1, pl.ds(j * 16, 16)]
            plsc.addupdate_scatter(hist_v, [dv], ones16)

    def step(k, j):
        ib, b = j, j % 2
        gather(ib, b).wait()
        scatter(ib, b).start(add=True)
        hist(ib)

        @pl.when(k >= 1)
        def _():
            scatter((j + 3) % 4, (j + 1) % 2).wait()

        @pl.when(k + 1 < NCHW)
        def _():
            idxcopy(k + 1, (j + 1) % 4).wait()
            gather((j + 1) % 4, (j + 1) % 2).start()

        @pl.when(k + 2 < NCHW)
        def _():
            idxcopy(k + 2, (j + 2) % 4).start()

    def mbody(kk, _):
        for j in range(4):
            step(4 * kk + j, j)
        return 0
    lax.fori_loop(0, NCHW // 4, mbody, 0)
    step(NCHW - 2, (NCHW - 2) % 4)
    step(NCHW - 1, (NCHW - 1) % 4)

    scatter((NCHW - 1) % 4, (NCHW - 1) % 2).wait()

    @pl.when(wid < NEXTRA)
    def _():
        ke = NW * NCHW + wid - c0
        idxcopy(ke, 0).start()
        idxcopy(ke, 0).wait()
        gather(0, 0).start()
        gather(0, 0).wait()
        scatter(0, 0).start(add=True)
        hist(0)
        scatter(0, 0).wait()

    pltpu.sync_copy(hist_v, deg_hbm.at[wid])

    plsc.subcore_barrier()

    def ofire(k, _):
        r = row0 + k * CPROWS
        pltpu.async_copy(acc_sh.at[pl.ds(r, CPROWS)],
                         out_hbm.at[cid, pl.ds(r, CPROWS)], zsem)
        return 0
    lax.fori_loop(0, RPW // CPROWS, ofire, 0)

    def odrain(k, _):
        r = row0 + k * CPROWS
        pltpu.make_async_copy(acc_sh.at[pl.ds(r, CPROWS)],
                              out_hbm.at[cid, pl.ds(r, CPROWS)], zsem).wait()
        return 0
    lax.fori_loop(0, RPW // CPROWS, odrain, 0)


def _sc_agg(tangent, edge_index):
    mesh = plsc.VectorSubcoreMesh(core_axis_name="c", subcore_axis_name="s")
    f = functools.partial(
        pl.kernel,
        out_type=(
            jax.ShapeDtypeStruct((NC, NPAD, D), jnp.float32),
            jax.ShapeDtypeStruct((NW, NPAD), jnp.float32),
        ),
        mesh=mesh,
        scratch_types=[
            [pltpu.VMEM((2, CH), jnp.int32) for _ in range(4)],
            [pltpu.VMEM((CH, D), jnp.float32) for _ in range(2)],
            pltpu.VMEM((ZROWS, D), jnp.float32),
            pltpu.VMEM((NPAD,), jnp.float32),
            pltpu.VMEM_SHARED((NPAD, D), jnp.float32),
            [pltpu.SemaphoreType.DMA for _ in range(4)],
            [pltpu.SemaphoreType.DMA for _ in range(2)],
            [pltpu.SemaphoreType.DMA for _ in range(2)],
            pltpu.SemaphoreType.DMA,
        ],
        compiler_params=pltpu.CompilerParams(needs_layout_passes=False),
    )(_sc_agg_body)
    return f(tangent, edge_index)


def _tc_post_body(p_ref, deg_ref, w_ref, b_ref, c_ref, o_ref):
    agg_t = p_ref[0] + p_ref[1]
    ones = jnp.ones((NW, 1), jnp.float32)
    deg = lax.dot_general(
        deg_ref[...], ones,
        dimension_numbers=(((0,), (0,)), ((), ())),
        preferred_element_type=jnp.float32,
        precision=lax.Precision.HIGHEST,
    )
    agg = lax.dot_general(
        agg_t, w_ref[...],
        dimension_numbers=(((1,), (1,)), ((), ())),
        preferred_element_type=jnp.float32,
        precision=lax.Precision.HIGHEST,
    ) + deg * b_ref[...].reshape(1, D)
    neigh = agg / jnp.maximum(deg, 1.0)
    vn = jnp.sqrt(jnp.sum(neigh * neigh, axis=1, keepdims=True))
    vn = jnp.maximum(vn, 1e-15)
    cs = jnp.sqrt(jnp.abs(c_ref[...]))
    arg = cs * vn
    o_ref[...] = jnp.tanh(arg) * neigh / arg


def _tc_post(partials, degs, W, b, c_param):
    br = 1024
    return pl.pallas_call(
        _tc_post_body,
        grid=(NPAD // br,),
        in_specs=[
            pl.BlockSpec((NC, br, D), lambda i: (0, i, 0)),
            pl.BlockSpec((NW, br), lambda i: (0, i)),
            pl.BlockSpec((D, D), lambda i: (0, 0)),
            pl.BlockSpec((D,), lambda i: (0,)),
            pl.BlockSpec((1,), lambda i: (0,)),
        ],
        out_specs=pl.BlockSpec((br, D), lambda i: (i, 0)),
        out_shape=jax.ShapeDtypeStruct((N, D), jnp.float32),
    )(partials, degs, W, b, c_param)


def kernel(x, edge_index, W, b, c_param):
    tangent = _tc_pre(x, c_param)
    partials, degs = _sc_agg(tangent, edge_index)
    return _tc_post(partials, degs, W, b, c_param)


# speedup vs baseline: 1.2513x; 1.1481x over previous
import functools

import jax
import jax.numpy as jnp
from jax import lax
from jax.experimental import pallas as pl
from jax.experimental.pallas import tpu as pltpu
from jax.experimental.pallas import tpu_sc as plsc

N = 10000
D = 128
E = 320000
NC = 2
NS = 16
NW = NC * NS
EPW = E // NW
CH = 128
NCHT = E // CH
NCHW = NCHT // NW
NEXTRA = NCHT - NCHW * NW
NPAD = 10240
RPW = NPAD // NS
ZROWS = 40
CPROWS = 160


def _tc_pre_body(x_ref, c_ref, o_ref):
    x = x_ref[...]
    cs = jnp.sqrt(jnp.abs(c_ref[...]))
    xn = jnp.sqrt(jnp.sum(x * x, axis=1, keepdims=True))
    xn = jnp.maximum(xn, 1e-15)
    y = jnp.clip(cs * xn, -1.0 + 1e-7, 1.0 - 1e-7)
    at = 0.5 * jnp.log((1.0 + y) / (1.0 - y))
    o_ref[...] = at * x / (cs * xn)


def _tc_pre(x, c_param):
    br = 1000
    return pl.pallas_call(
        _tc_pre_body,
        grid=(N // br,),
        in_specs=[
            pl.BlockSpec((br, D), lambda i: (i, 0)),
            pl.BlockSpec((1,), lambda i: (0,)),
        ],
        out_specs=pl.BlockSpec((br, D), lambda i: (i, 0)),
        out_shape=jax.ShapeDtypeStruct((N, D), jnp.float32),
    )(x, c_param)


def _sc_agg_body(tang_hbm, edge_hbm, out_hbm, deg_hbm,
                 idxb, rowsb, zero_v, hist_v, acc_sh,
                 isems, gsems, ssems, zsem):
    cid = lax.axis_index("c")
    sid = lax.axis_index("s")
    wid = cid * NS + sid
    row0 = sid * RPW
    c0 = wid * NCHW

    def idxcopy(k, ib):
        off = pl.multiple_of((c0 + k) * CH, CH)
        return pltpu.make_async_copy(edge_hbm.at[:, pl.ds(off, CH)],
                                     idxb[ib], isems[ib])

    def gather(ib, b):
        return pltpu.make_async_copy(tang_hbm.at[idxb[ib].at[0]],
                                     rowsb[b], gsems[b])

    def scatter(ib, b):
        return pltpu.make_async_copy(rowsb[b], acc_sh.at[idxb[ib].at[1]],
                                     ssems[b])

    idxcopy(0, 0).start()
    idxcopy(1, 1).start()
    idxcopy(0, 0).wait()
    gather(0, 0).start()

    def zbody(i, _):
        r = i // (D // 16)
        c0 = (i % (D // 16)) * 16
        zero_v[r, pl.ds(c0, 16)] = jnp.zeros((16,), jnp.float32)
        return 0
    lax.fori_loop(0, ZROWS * (D // 16), zbody, 0)

    def zfire(k, _):
        pltpu.async_copy(zero_v, acc_sh.at[pl.ds(row0 + k * ZROWS, ZROWS)],
                         zsem)
        return 0
    lax.fori_loop(0, RPW // ZROWS, zfire, 0)

    def hzero(i, _):
        hist_v[pl.ds(i * 16, 16)] = jnp.zeros((16,), jnp.float32)
        return 0
    lax.fori_loop(0, NPAD // 16, hzero, 0)

    def zdrain(k, _):
        pltpu.make_async_copy(
            zero_v, acc_sh.at[pl.ds(row0 + k * ZROWS, ZROWS)], zsem).wait()
        return 0
    lax.fori_loop(0, RPW // ZROWS, zdrain, 0)

    plsc.subcore_barrier()

    ones16 = jnp.ones((16,), jnp.float32)

    def hist(ib):
        for j in range(CH // 16):
            dv = idxb[ib][1, pl.ds(j * 16, 16)]
            plsc.addupdate_scatter(hist_v, [dv], ones16)

    def step(k, j):
        ib, b = j, j % 2

        @pl.when(k >= 1)
        def _():
            scatter((j + 3) % 4, (j + 1) % 2).wait()

        @pl.when(k + 1 < NCHW)
        def _():
            idxcopy(k + 1, (j + 1) % 4).wait()
            gather((j + 1) % 4, (j + 1) % 2).start()

        gather(ib, b).wait()
        scatter(ib, b).start(add=True)
        hist(ib)

        @pl.when(k + 2 < NCHW)
        def _():
            idxcopy(k + 2, (j + 2) % 4).start()

    def mbody(kk, _):
        for j in range(4):
            step(4 * kk + j, j)
        return 0
    lax.fori_loop(0, NCHW // 4, mbody, 0)
    step(NCHW - 2, (NCHW - 2) % 4)
    step(NCHW - 1, (NCHW - 1) % 4)

    scatter((NCHW - 1) % 4, (NCHW - 1) % 2).wait()

    @pl.when(wid < NEXTRA)
    def _():
        ke = NW * NCHW + wid - c0
        idxcopy(ke, 0).start()
        idxcopy(ke, 0).wait()
        gather(0, 0).start()
        gather(0, 0).wait()
        scatter(0, 0).start(add=True)
        hist(0)
        scatter(0, 0).wait()

    pltpu.sync_copy(hist_v, deg_hbm.at[wid])

    plsc.subcore_barrier()

    def ofire(k, _):
        r = row0 + k * CPROWS
        pltpu.async_copy(acc_sh.at[pl.ds(r, CPROWS)],
                         out_hbm.at[cid, pl.ds(r, CPROWS)], zsem)
        return 0
    lax.fori_loop(0, RPW // CPROWS, ofire, 0)

    def odrain(k, _):
        r = row0 + k * CPROWS
        pltpu.make_async_copy(acc_sh.at[pl.ds(r, CPROWS)],
                              out_hbm.at[cid, pl.ds(r, CPROWS)], zsem).wait()
        return 0
    lax.fori_loop(0, RPW // CPROWS, odrain, 0)


def _sc_agg(tangent, edge_index):
    mesh = plsc.VectorSubcoreMesh(core_axis_name="c", subcore_axis_name="s")
    f = functools.partial(
        pl.kernel,
        out_type=(
            jax.ShapeDtypeStruct((NC, NPAD, D), jnp.float32),
            jax.ShapeDtypeStruct((NW, NPAD), jnp.float32),
        ),
        mesh=mesh,
        scratch_types=[
            [pltpu.VMEM((2, CH), jnp.int32) for _ in range(4)],
            [pltpu.VMEM((CH, D), jnp.float32) for _ in range(2)],
            pltpu.VMEM((ZROWS, D), jnp.float32),
            pltpu.VMEM((NPAD,), jnp.float32),
            pltpu.VMEM_SHARED((NPAD, D), jnp.float32),
            [pltpu.SemaphoreType.DMA for _ in range(4)],
            [pltpu.SemaphoreType.DMA for _ in range(2)],
            [pltpu.SemaphoreType.DMA for _ in range(2)],
            pltpu.SemaphoreType.DMA,
        ],
        compiler_params=pltpu.CompilerParams(needs_layout_passes=False),
    )(_sc_agg_body)
    return f(tangent, edge_index)


def _tc_post_body(p_ref, deg_ref, w_ref, b_ref, c_ref, o_ref):
    agg_t = p_ref[0] + p_ref[1]
    ones = jnp.ones((NW, 1), jnp.float32)
    deg = lax.dot_general(
        deg_ref[...], ones,
        dimension_numbers=(((0,), (0,)), ((), ())),
        preferred_element_type=jnp.float32,
        precision=lax.Precision.HIGHEST,
    )
    agg = lax.dot_general(
        agg_t, w_ref[...],
        dimension_numbers=(((1,), (1,)), ((), ())),
        preferred_element_type=jnp.float32,
        precision=lax.Precision.HIGHEST,
    ) + deg * b_ref[...].reshape(1, D)
    neigh = agg / jnp.maximum(deg, 1.0)
    vn = jnp.sqrt(jnp.sum(neigh * neigh, axis=1, keepdims=True))
    vn = jnp.maximum(vn, 1e-15)
    cs = jnp.sqrt(jnp.abs(c_ref[...]))
    arg = cs * vn
    o_ref[...] = jnp.tanh(arg) * neigh / arg


def _tc_post(partials, degs, W, b, c_param):
    br = 1024
    return pl.pallas_call(
        _tc_post_body,
        grid=(NPAD // br,),
        in_specs=[
            pl.BlockSpec((NC, br, D), lambda i: (0, i, 0)),
            pl.BlockSpec((NW, br), lambda i: (0, i)),
            pl.BlockSpec((D, D), lambda i: (0, 0)),
            pl.BlockSpec((D,), lambda i: (0,)),
            pl.BlockSpec((1,), lambda i: (0,)),
        ],
        out_specs=pl.BlockSpec((br, D), lambda i: (i, 0)),
        out_shape=jax.ShapeDtypeStruct((N, D), jnp.float32),
    )(partials, degs, W, b, c_param)


def kernel(x, edge_index, W, b, c_param):
    tangent = _tc_pre(x, c_param)
    partials, degs = _sc_agg(tangent, edge_index)
    return _tc_post(partials, degs, W, b, c_param)
